# Initial kernel scaffold; baseline (speedup 1.0000x reference)
import jax
import jax.numpy as jnp
from jax.experimental import pallas as pl


def kernel(x, cluster_ids, W0, b0, W1, b1, W2, b2, W3, b3, W4, b4):
    raise NotImplementedError("write your pallas kernel here")


# speedup vs baseline: 63.3069x; 63.3069x over previous
import functools
import math

import jax
import jax.numpy as jnp
import numpy as np
from jax import lax
from jax.experimental import pallas as pl
from jax.experimental.pallas import tpu as pltpu
from jax.experimental.pallas import tpu_sc as plsc

N_TOKENS = 131072
NUM_CLUSTERS = 16
HIDDEN = 256
OUT_DIM = 56
PE_LEVELS = 10
IN_DIM = 3 + 3 * 2 * PE_LEVELS
DW = 128

B_T = 256
N_PAD = N_TOKENS + NUM_CLUSTERS * B_T
NB = N_PAD // B_T
NB_PAD = ((NB + 15) // 16) * 16

NW = 32
CHUNK = N_TOKENS // NW
CROWS = CHUNK // DW

@functools.lru_cache(maxsize=None)
def _sc_mesh_opts():
  return dict(
      mesh=plsc.VectorSubcoreMesh(core_axis_name="c", subcore_axis_name="s"),
      compiler_params=pltpu.CompilerParams(needs_layout_passes=False),
  )


def _wid():
  return lax.axis_index("s") * 2 + lax.axis_index("c")


def _pe_body(x_ref, out_ref):
  xb = x_ref[...]
  rows = xb.shape[0]
  col = lax.broadcasted_iota(jnp.int32, (rows, DW), 1)
  col1 = lax.broadcasted_iota(jnp.int32, (3, DW), 1)
  drow = lax.broadcasted_iota(jnp.int32, (3, DW), 0)
  r = (col1 - 3) % 6
  dmap = jnp.where(col1 < 3, col1, r % 3)
  sel = jnp.where(dmap == drow, 1.0, 0.0).astype(jnp.float32)
  X = jnp.dot(xb, sel, preferred_element_type=jnp.float32)
  lvl = lax.broadcasted_iota(jnp.int32, (rows, DW), 1)
  lvl = jnp.maximum(lvl - 3, 0) // 6
  freq = jnp.exp2(lvl.astype(jnp.float32)) * math.pi
  is_cos = ((col - 3) % 6) >= 3
  phase = jnp.where(is_cos, math.pi / 2.0, 0.0).astype(jnp.float32)
  Z = jnp.sin(X * freq + phase)
  out_ref[...] = jnp.where(col < 3, X, jnp.where(col < IN_DIM, Z, 0.0))


def _pe_encode(x):
  rows = 1024
  return pl.pallas_call(
      _pe_body,
      grid=(N_TOKENS // rows,),
      in_specs=[pl.BlockSpec((rows, 3), lambda b: (b, 0))],
      out_specs=pl.BlockSpec((rows, DW), lambda b: (b, 0)),
      out_shape=jax.ShapeDtypeStruct((N_TOKENS, DW), jnp.float32),
  )(x)


@functools.lru_cache(maxsize=None)
def _make_hist_kernel():
  return pl.kernel(
      _hist_body,
      out_type=jax.ShapeDtypeStruct((NW, 16), jnp.int32),
      scratch_types=[
          pltpu.VMEM((CHUNK,), jnp.int32),
          pltpu.VMEM((16,), jnp.int32),
      ],
      **_sc_mesh_opts(),
  )


def _hist_body(ids_hbm, hist_o, ids_v, hist_v):
  wid = _wid()
  pltpu.sync_copy(ids_hbm.at[pl.ds(wid * CHUNK, CHUNK)], ids_v)
  hist_v[...] = jnp.zeros((16,), jnp.int32)
  ones = jnp.ones((16,), jnp.int32)

  def body(k, _):
    c = ids_v[pl.ds(k * 16, 16)]
    plsc.addupdate_scatter(hist_v, [c], ones)
    return 0

  lax.fori_loop(0, CHUNK // 16, body, 0)
  pltpu.sync_copy(hist_v, hist_o.at[wid])


@functools.lru_cache(maxsize=None)
def _make_route_kernel():
  return pl.kernel(
      _route_body,
      out_type=(
          jax.ShapeDtypeStruct((N_TOKENS // DW, DW), jnp.int32),
          jax.ShapeDtypeStruct((N_PAD, DW), jnp.float32),
          jax.ShapeDtypeStruct((NB_PAD,), jnp.int32),
      ),
      scratch_types=[
          pltpu.VMEM((CHUNK,), jnp.int32),
          pltpu.VMEM((NW, 16), jnp.int32),
          pltpu.VMEM((16,), jnp.int32),
          pltpu.VMEM((16,), jnp.int32),
          pltpu.VMEM((16,), jnp.int32),
          pltpu.VMEM((16,), jnp.int32),
          pltpu.VMEM((CROWS, DW), jnp.int32),
          pltpu.VMEM((NB_PAD,), jnp.int32),
          pltpu.VMEM((DW, DW), jnp.float32),
          pltpu.VMEM((DW, DW), jnp.float32),
          pltpu.SemaphoreType.DMA,
          pltpu.SemaphoreType.DMA,
      ],
      **_sc_mesh_opts(),
  )


def _route_body(ids_hbm, hist_hbm, xe_hbm, dpos_o, xs_o, bexp_o,
                  ids_v, hist_v, base_v, seg_v, tmp_v, tmp2_v, pos_v, bexp_v,
                  bufa, bufb, sema, semb):
  wid = _wid()
  i16 = lax.iota(jnp.int32, 16)
  ones = jnp.ones((16,), jnp.int32)
  zeros = jnp.zeros((16,), jnp.int32)

  pltpu.sync_copy(ids_hbm.at[pl.ds(wid * CHUNK, CHUNK)], ids_v)
  pltpu.sync_copy(hist_hbm, hist_v)

  totals = zeros
  pre = zeros
  for t in range(NW):
    row = hist_v[t]
    totals = totals + row
    pre = pre + jnp.where(t < wid, row, zeros)
  pt = (totals + (B_T - 1)) & ~(B_T - 1)
  seg = plsc.cumsum(pt) - pt
  base_v[...] = seg + pre
  seg_v[...] = seg

  def body(k, _):
    c = ids_v[pl.ds(k * 16, 16)]
    s, v = plsc.sort_key_val(c, i16)
    tmp_v[...] = s
    sh = plsc.load_gather(tmp_v, [jnp.maximum(i16 - 1, 0)])
    bnd = (i16 == 0) | (s != sh)
    st = plsc.cummax(jnp.where(bnd, i16, 0))
    plsc.store_scatter(tmp2_v, [v], i16 - st)
    rank = tmp2_v[...]
    g = plsc.load_gather(base_v, [c])
    pos = g + rank
    plsc.addupdate_scatter(base_v, [c], ones)
    pos_v[k // 8, pl.ds((k % 8) * 16, 16)] = pos
    return 0

  lax.fori_loop(0, CHUNK // 16, body, 0)

  pltpu.sync_copy(pos_v, dpos_o.at[pl.ds(wid * CROWS, CROWS)])

  row0 = wid * CHUNK
  pltpu.sync_copy(xe_hbm.at[pl.ds(row0, DW)], bufa)
  pltpu.async_copy(bufa, xs_o.at[pos_v.at[0]], sema)

  def sbody(j, _):
    @pl.when(j % 2 == 0)
    def _():
      pltpu.sync_copy(xe_hbm.at[pl.ds(row0 + (j + 1) * DW, DW)], bufb)
      pltpu.async_copy(bufb, xs_o.at[pos_v.at[j + 1]], semb)
      pltpu.make_async_copy(bufa, xs_o.at[pos_v.at[j]], sema).wait()

    @pl.when(j % 2 == 1)
    def _():
      pltpu.sync_copy(xe_hbm.at[pl.ds(row0 + (j + 1) * DW, DW)], bufa)
      pltpu.async_copy(bufa, xs_o.at[pos_v.at[j + 1]], sema)
      pltpu.make_async_copy(bufb, xs_o.at[pos_v.at[j]], semb).wait()

    return 0

  lax.fori_loop(0, CROWS - 1, sbody, 0)
  pltpu.make_async_copy(bufb, xs_o.at[pos_v.at[CROWS - 1]], semb).wait()

  @pl.when(wid == 0)
  def _():
    def bbody(i, _):
      p16 = i16 + i * 16
      acc = jnp.full((16,), -1, jnp.int32)
      for cl in range(NUM_CLUSTERS):
        sv = plsc.load_gather(seg_v, [jnp.full((16,), cl, jnp.int32)])
        acc = acc + jnp.where(p16 * B_T >= sv, 1, 0)
      bexp_v[pl.ds(i * 16, 16)] = acc
      return 0

    lax.fori_loop(0, NB_PAD // 16, bbody, 0)
    pltpu.sync_copy(bexp_v, bexp_o)


def _mlp_body(bexp_ref, xs_ref, w0, b0, w1, b1, w2, b2, w3, b3, w4, b4,
              out_ref):
  del bexp_ref
  f32 = jnp.float32
  h = xs_ref[...]
  h = jnp.tanh(jnp.dot(h, w0[0], preferred_element_type=f32) + b0[0])
  h = jnp.tanh(jnp.dot(h, w1[0], preferred_element_type=f32) + b1[0])
  h = jnp.tanh(jnp.dot(h, w2[0], preferred_element_type=f32) + b2[0])
  h = jnp.tanh(jnp.dot(h, w3[0], preferred_element_type=f32) + b3[0])
  out_ref[...] = jnp.dot(h, w4[0], preferred_element_type=f32) + b4[0]


def _grouped_mlp(xs, bexp, w0p, b0r, w1, b1r, w2, b2r, w3, b3r, w4p, b4r):
  def xmap(b, bexp_ref):
    del bexp_ref
    return (b, 0)

  def wmap(b, bexp_ref):
    return (bexp_ref[b], 0, 0)

  grid_spec = pltpu.PrefetchScalarGridSpec(
      num_scalar_prefetch=1,
      grid=(NB,),
      in_specs=[
          pl.BlockSpec((B_T, DW), xmap),
          pl.BlockSpec((1, DW, HIDDEN), wmap),
          pl.BlockSpec((1, 1, HIDDEN), wmap),
          pl.BlockSpec((1, HIDDEN, HIDDEN), wmap),
          pl.BlockSpec((1, 1, HIDDEN), wmap),
          pl.BlockSpec((1, HIDDEN, HIDDEN), wmap),
          pl.BlockSpec((1, 1, HIDDEN), wmap),
          pl.BlockSpec((1, HIDDEN, HIDDEN), wmap),
          pl.BlockSpec((1, 1, HIDDEN), wmap),
          pl.BlockSpec((1, HIDDEN, DW), wmap),
          pl.BlockSpec((1, 1, DW), wmap),
      ],
      out_specs=pl.BlockSpec((B_T, DW), xmap),
  )
  return pl.pallas_call(
      _mlp_body,
      grid_spec=grid_spec,
      out_shape=jax.ShapeDtypeStruct((N_PAD, DW), jnp.float32),
      compiler_params=pltpu.CompilerParams(
          dimension_semantics=("arbitrary",),
      ),
  )(bexp, xs, w0p, b0r, w1, b1r, w2, b2r, w3, b3r, w4p, b4r)


@functools.lru_cache(maxsize=None)
def _make_gather_kernel():
  return pl.kernel(
      _gather_body,
      out_type=jax.ShapeDtypeStruct((N_TOKENS, DW), jnp.float32),
      scratch_types=[
          pltpu.VMEM((CROWS, DW), jnp.int32),
          pltpu.VMEM((DW, DW), jnp.float32),
          pltpu.VMEM((DW, DW), jnp.float32),
          pltpu.SemaphoreType.DMA,
          pltpu.SemaphoreType.DMA,
      ],
      **_sc_mesh_opts(),
  )


def _gather_body(dpos_hbm, outs_hbm, outw_o, pos_v, bufa, bufb, sema, semb):
  wid = _wid()
  pltpu.sync_copy(dpos_hbm.at[pl.ds(wid * CROWS, CROWS)], pos_v)
  row0 = wid * CHUNK

  cpa = pltpu.async_copy(outs_hbm.at[pos_v.at[0]], bufa, sema)

  def gbody2(j, _):
    @pl.when(j % 2 == 0)
    def _():
      cpb = pltpu.async_copy(outs_hbm.at[pos_v.at[j + 1]], bufb, semb)
      pltpu.sync_copy(bufa, outw_o.at[pl.ds(row0 + j * DW, DW)])
      cpb.wait()

    @pl.when(j % 2 == 1)
    def _():
      cpa2 = pltpu.async_copy(outs_hbm.at[pos_v.at[j + 1]], bufa, sema)
      pltpu.sync_copy(bufb, outw_o.at[pl.ds(row0 + j * DW, DW)])
      cpa2.wait()

    return 0

  cpa.wait()
  lax.fori_loop(0, CROWS - 1, gbody2, 0)
  last = CROWS - 1
  buf_last = bufa if (last % 2 == 0) else bufb
  pltpu.sync_copy(buf_last, outw_o.at[pl.ds(row0 + last * DW, DW)])


def _narrow_body(in_ref, out_ref):
  out_ref[...] = in_ref[:, :OUT_DIM]


def _narrow(outw):
  rows = 1024
  return pl.pallas_call(
      _narrow_body,
      grid=(N_TOKENS // rows,),
      in_specs=[pl.BlockSpec((rows, DW), lambda b: (b, 0))],
      out_specs=pl.BlockSpec((rows, OUT_DIM), lambda b: (b, 0)),
      out_shape=jax.ShapeDtypeStruct((N_TOKENS, OUT_DIM), jnp.float32),
  )(outw)


def kernel(x, cluster_ids, W0, b0, W1, b1, W2, b2, W3, b3, W4, b4):
  ids = cluster_ids.astype(jnp.int32)

  w0p = jnp.pad(W0, ((0, 0), (0, DW - IN_DIM), (0, 0)))
  w4p = jnp.pad(W4, ((0, 0), (0, 0), (0, DW - OUT_DIM)))
  b4p = jnp.pad(b4, ((0, 0), (0, DW - OUT_DIM)))
  b0r = b0.reshape(NUM_CLUSTERS, 1, HIDDEN)
  b1r = b1.reshape(NUM_CLUSTERS, 1, HIDDEN)
  b2r = b2.reshape(NUM_CLUSTERS, 1, HIDDEN)
  b3r = b3.reshape(NUM_CLUSTERS, 1, HIDDEN)
  b4r = b4p.reshape(NUM_CLUSTERS, 1, DW)

  xe = _pe_encode(x)
  hist = _make_hist_kernel()(ids)
  dpos, xs, bexp = _make_route_kernel()(ids, hist, xe)
  outs = _grouped_mlp(xs, bexp, w0p, b0r, W1, b1r, W2, b2r, W3, b3r, w4p, b4r)
  outw = _make_gather_kernel()(dpos, outs)
  return _narrow(outw)


# speedup vs baseline: 68.7838x; 1.0865x over previous
import functools
import math

import jax
import jax.numpy as jnp
import numpy as np
from jax import lax
from jax.experimental import pallas as pl
from jax.experimental.pallas import tpu as pltpu
from jax.experimental.pallas import tpu_sc as plsc

N_TOKENS = 131072
NUM_CLUSTERS = 16
HIDDEN = 256
OUT_DIM = 56
PE_LEVELS = 10
IN_DIM = 3 + 3 * 2 * PE_LEVELS
DW = 128

B_T = 256
N_PAD = N_TOKENS + NUM_CLUSTERS * B_T
NB = N_PAD // B_T
NB_PAD = ((NB + 15) // 16) * 16

NW = 32
CHUNK = N_TOKENS // NW
CROWS = CHUNK // DW

@functools.lru_cache(maxsize=None)
def _sc_mesh_opts():
  return dict(
      mesh=plsc.VectorSubcoreMesh(core_axis_name="c", subcore_axis_name="s"),
      compiler_params=pltpu.CompilerParams(needs_layout_passes=False),
  )


def _wid():
  return lax.axis_index("s") * 2 + lax.axis_index("c")


def _pe_body(x_ref, out_ref):
  xb = x_ref[...]
  rows = xb.shape[0]
  col = lax.broadcasted_iota(jnp.int32, (rows, DW), 1)
  col1 = lax.broadcasted_iota(jnp.int32, (3, DW), 1)
  drow = lax.broadcasted_iota(jnp.int32, (3, DW), 0)
  r = (col1 - 3) % 6
  dmap = jnp.where(col1 < 3, col1, r % 3)
  sel = jnp.where(dmap == drow, 1.0, 0.0).astype(jnp.float32)
  X = jnp.dot(xb, sel, preferred_element_type=jnp.float32)
  lvl = lax.broadcasted_iota(jnp.int32, (rows, DW), 1)
  lvl = jnp.maximum(lvl - 3, 0) // 6
  freqpow = jnp.exp2(lvl.astype(jnp.float32))
  is_cos = ((col - 3) % 6) >= 3
  phase = jnp.where(is_cos, 0.5, 0.0).astype(jnp.float32)
  u = X - 2.0 * jnp.round(X * 0.5)
  f = u * freqpow
  w = f + phase
  t = w - 2.0 * jnp.round(w * 0.5)
  a = jnp.abs(t)
  m = jnp.minimum(a, 1.0 - a)
  s = jnp.where(t < 0, -m, m)
  z = s * s
  p = jnp.float32(-7.3704309e-03)
  p = p * z + jnp.float32(8.2145887e-02)
  p = p * z - jnp.float32(5.9926453e-01)
  p = p * z + jnp.float32(2.5501640e+00)
  p = p * z - jnp.float32(5.1677128e+00)
  p = p * z + jnp.float32(3.1415927e+00)
  Z = s * p
  out_ref[...] = jnp.where(col < 3, X, jnp.where(col < IN_DIM, Z, 0.0))


def _pe_encode(x):
  rows = 1024
  return pl.pallas_call(
      _pe_body,
      grid=(N_TOKENS // rows,),
      in_specs=[pl.BlockSpec((rows, 3), lambda b: (b, 0))],
      out_specs=pl.BlockSpec((rows, DW), lambda b: (b, 0)),
      out_shape=jax.ShapeDtypeStruct((N_TOKENS, DW), jnp.float32),
  )(x)


@functools.lru_cache(maxsize=None)
def _make_hist_kernel():
  return pl.kernel(
      _hist_body,
      out_type=jax.ShapeDtypeStruct((NW, 16), jnp.int32),
      scratch_types=[
          pltpu.VMEM((CHUNK,), jnp.int32),
          pltpu.VMEM((16,), jnp.int32),
      ],
      **_sc_mesh_opts(),
  )


def _hist_body(ids_hbm, hist_o, ids_v, hist_v):
  wid = _wid()
  pltpu.sync_copy(ids_hbm.at[pl.ds(wid * CHUNK, CHUNK)], ids_v)
  hist_v[...] = jnp.zeros((16,), jnp.int32)
  ones = jnp.ones((16,), jnp.int32)

  def body(k, _):
    c = ids_v[pl.ds(k * 16, 16)]
    plsc.addupdate_scatter(hist_v, [c], ones)
    return 0

  lax.fori_loop(0, CHUNK // 16, body, 0)
  pltpu.sync_copy(hist_v, hist_o.at[wid])


@functools.lru_cache(maxsize=None)
def _make_route_kernel():
  return pl.kernel(
      _route_body,
      out_type=(
          jax.ShapeDtypeStruct((N_TOKENS // DW, DW), jnp.int32),
          jax.ShapeDtypeStruct((N_PAD, DW), jnp.float32),
          jax.ShapeDtypeStruct((NB_PAD,), jnp.int32),
      ),
      scratch_types=[
          pltpu.VMEM((CHUNK,), jnp.int32),
          pltpu.VMEM((NW, 16), jnp.int32),
          pltpu.VMEM((16,), jnp.int32),
          pltpu.VMEM((16,), jnp.int32),
          pltpu.VMEM((16,), jnp.int32),
          pltpu.VMEM((16,), jnp.int32),
          pltpu.VMEM((CROWS, DW), jnp.int32),
          pltpu.VMEM((NB_PAD,), jnp.int32),
          pltpu.VMEM((DW, DW), jnp.float32),
          pltpu.VMEM((DW, DW), jnp.float32),
          pltpu.SemaphoreType.DMA,
          pltpu.SemaphoreType.DMA,
      ],
      **_sc_mesh_opts(),
  )


def _route_body(ids_hbm, hist_hbm, xe_hbm, dpos_o, xs_o, bexp_o,
                  ids_v, hist_v, base_v, seg_v, tmp_v, tmp2_v, pos_v, bexp_v,
                  bufa, bufb, sema, semb):
  wid = _wid()
  i16 = lax.iota(jnp.int32, 16)
  ones = jnp.ones((16,), jnp.int32)
  zeros = jnp.zeros((16,), jnp.int32)

  pltpu.sync_copy(ids_hbm.at[pl.ds(wid * CHUNK, CHUNK)], ids_v)
  pltpu.sync_copy(hist_hbm, hist_v)

  totals = zeros
  pre = zeros
  for t in range(NW):
    row = hist_v[t]
    totals = totals + row
    pre = pre + jnp.where(t < wid, row, zeros)
  pt = (totals + (B_T - 1)) & ~(B_T - 1)
  seg = plsc.cumsum(pt) - pt
  base_v[...] = seg + pre
  seg_v[...] = seg

  def body(k, _):
    c = ids_v[pl.ds(k * 16, 16)]
    s, v = plsc.sort_key_val(c, i16)
    tmp_v[...] = s
    sh = plsc.load_gather(tmp_v, [jnp.maximum(i16 - 1, 0)])
    bnd = (i16 == 0) | (s != sh)
    st = plsc.cummax(jnp.where(bnd, i16, 0))
    plsc.store_scatter(tmp2_v, [v], i16 - st)
    rank = tmp2_v[...]
    g = plsc.load_gather(base_v, [c])
    pos = g + rank
    plsc.addupdate_scatter(base_v, [c], ones)
    pos_v[k // 8, pl.ds((k % 8) * 16, 16)] = pos
    return 0

  lax.fori_loop(0, CHUNK // 16, body, 0)

  pltpu.sync_copy(pos_v, dpos_o.at[pl.ds(wid * CROWS, CROWS)])

  row0 = wid * CHUNK
  pltpu.sync_copy(xe_hbm.at[pl.ds(row0, DW)], bufa)
  pltpu.async_copy(bufa, xs_o.at[pos_v.at[0]], sema)

  def sbody(j, _):
    @pl.when(j % 2 == 0)
    def _():
      pltpu.sync_copy(xe_hbm.at[pl.ds(row0 + (j + 1) * DW, DW)], bufb)
      pltpu.async_copy(bufb, xs_o.at[pos_v.at[j + 1]], semb)
      pltpu.make_async_copy(bufa, xs_o.at[pos_v.at[j]], sema).wait()

    @pl.when(j % 2 == 1)
    def _():
      pltpu.sync_copy(xe_hbm.at[pl.ds(row0 + (j + 1) * DW, DW)], bufa)
      pltpu.async_copy(bufa, xs_o.at[pos_v.at[j + 1]], sema)
      pltpu.make_async_copy(bufb, xs_o.at[pos_v.at[j]], semb).wait()

    return 0

  lax.fori_loop(0, CROWS - 1, sbody, 0)
  pltpu.make_async_copy(bufb, xs_o.at[pos_v.at[CROWS - 1]], semb).wait()

  @pl.when(wid == 0)
  def _():
    def bbody(i, _):
      p16 = i16 + i * 16
      acc = jnp.full((16,), -1, jnp.int32)
      for cl in range(NUM_CLUSTERS):
        sv = plsc.load_gather(seg_v, [jnp.full((16,), cl, jnp.int32)])
        acc = acc + jnp.where(p16 * B_T >= sv, 1, 0)
      bexp_v[pl.ds(i * 16, 16)] = acc
      return 0

    lax.fori_loop(0, NB_PAD // 16, bbody, 0)
    pltpu.sync_copy(bexp_v, bexp_o)


def _mlp_body(bexp_ref, xs_ref, w0, b0, w1, b1, w2, b2, w3, b3, w4, b4,
              out_ref):
  del bexp_ref
  f32 = jnp.float32
  bf = jnp.bfloat16
  h = xs_ref[...].astype(bf)
  h = jnp.tanh(jnp.dot(h, w0[0], preferred_element_type=f32) + b0[0])
  h = jnp.tanh(jnp.dot(h.astype(bf), w1[0], preferred_element_type=f32) + b1[0])
  h = jnp.tanh(jnp.dot(h.astype(bf), w2[0], preferred_element_type=f32) + b2[0])
  h = jnp.tanh(jnp.dot(h.astype(bf), w3[0], preferred_element_type=f32) + b3[0])
  out_ref[...] = jnp.dot(h.astype(bf), w4[0], preferred_element_type=f32) + b4[0]


def _grouped_mlp(xs, bexp, w0p, b0r, w1, b1r, w2, b2r, w3, b3r, w4p, b4r):
  def xmap(b, bexp_ref):
    del bexp_ref
    return (b, 0)

  def wmap(b, bexp_ref):
    return (bexp_ref[b], 0, 0)

  grid_spec = pltpu.PrefetchScalarGridSpec(
      num_scalar_prefetch=1,
      grid=(NB,),
      in_specs=[
          pl.BlockSpec((B_T, DW), xmap),
          pl.BlockSpec((1, DW, HIDDEN), wmap),
          pl.BlockSpec((1, 1, HIDDEN), wmap),
          pl.BlockSpec((1, HIDDEN, HIDDEN), wmap),
          pl.BlockSpec((1, 1, HIDDEN), wmap),
          pl.BlockSpec((1, HIDDEN, HIDDEN), wmap),
          pl.BlockSpec((1, 1, HIDDEN), wmap),
          pl.BlockSpec((1, HIDDEN, HIDDEN), wmap),
          pl.BlockSpec((1, 1, HIDDEN), wmap),
          pl.BlockSpec((1, HIDDEN, DW), wmap),
          pl.BlockSpec((1, 1, DW), wmap),
      ],
      out_specs=pl.BlockSpec((B_T, DW), xmap),
  )
  return pl.pallas_call(
      _mlp_body,
      grid_spec=grid_spec,
      out_shape=jax.ShapeDtypeStruct((N_PAD, DW), jnp.float32),
      compiler_params=pltpu.CompilerParams(
          dimension_semantics=("arbitrary",),
      ),
  )(bexp, xs, w0p, b0r, w1, b1r, w2, b2r, w3, b3r, w4p, b4r)


@functools.lru_cache(maxsize=None)
def _make_gather_kernel():
  return pl.kernel(
      _gather_body,
      out_type=jax.ShapeDtypeStruct((N_TOKENS, DW), jnp.float32),
      scratch_types=[
          pltpu.VMEM((CROWS, DW), jnp.int32),
          pltpu.VMEM((DW, DW), jnp.float32),
          pltpu.VMEM((DW, DW), jnp.float32),
          pltpu.SemaphoreType.DMA,
          pltpu.SemaphoreType.DMA,
      ],
      **_sc_mesh_opts(),
  )


def _gather_body(dpos_hbm, outs_hbm, outw_o, pos_v, bufa, bufb, sema, semb):
  wid = _wid()
  pltpu.sync_copy(dpos_hbm.at[pl.ds(wid * CROWS, CROWS)], pos_v)
  row0 = wid * CHUNK

  cpa = pltpu.async_copy(outs_hbm.at[pos_v.at[0]], bufa, sema)

  def gbody2(j, _):
    @pl.when(j % 2 == 0)
    def _():
      cpb = pltpu.async_copy(outs_hbm.at[pos_v.at[j + 1]], bufb, semb)
      pltpu.sync_copy(bufa, outw_o.at[pl.ds(row0 + j * DW, DW)])
      cpb.wait()

    @pl.when(j % 2 == 1)
    def _():
      cpa2 = pltpu.async_copy(outs_hbm.at[pos_v.at[j + 1]], bufa, sema)
      pltpu.sync_copy(bufb, outw_o.at[pl.ds(row0 + j * DW, DW)])
      cpa2.wait()

    return 0

  cpa.wait()
  lax.fori_loop(0, CROWS - 1, gbody2, 0)
  last = CROWS - 1
  buf_last = bufa if (last % 2 == 0) else bufb
  pltpu.sync_copy(buf_last, outw_o.at[pl.ds(row0 + last * DW, DW)])


def _narrow_body(in_ref, out_ref):
  out_ref[...] = in_ref[:, :OUT_DIM]


def _narrow(outw):
  rows = 1024
  return pl.pallas_call(
      _narrow_body,
      grid=(N_TOKENS // rows,),
      in_specs=[pl.BlockSpec((rows, DW), lambda b: (b, 0))],
      out_specs=pl.BlockSpec((rows, OUT_DIM), lambda b: (b, 0)),
      out_shape=jax.ShapeDtypeStruct((N_TOKENS, OUT_DIM), jnp.float32),
  )(outw)


def kernel(x, cluster_ids, W0, b0, W1, b1, W2, b2, W3, b3, W4, b4):
  ids = cluster_ids.astype(jnp.int32)

  w0p = jnp.pad(W0, ((0, 0), (0, DW - IN_DIM), (0, 0))).astype(jnp.bfloat16)
  w4p = jnp.pad(W4, ((0, 0), (0, 0), (0, DW - OUT_DIM))).astype(jnp.bfloat16)
  w1c = W1.astype(jnp.bfloat16)
  w2c = W2.astype(jnp.bfloat16)
  w3c = W3.astype(jnp.bfloat16)
  b4p = jnp.pad(b4, ((0, 0), (0, DW - OUT_DIM)))
  b0r = b0.reshape(NUM_CLUSTERS, 1, HIDDEN)
  b1r = b1.reshape(NUM_CLUSTERS, 1, HIDDEN)
  b2r = b2.reshape(NUM_CLUSTERS, 1, HIDDEN)
  b3r = b3.reshape(NUM_CLUSTERS, 1, HIDDEN)
  b4r = b4p.reshape(NUM_CLUSTERS, 1, DW)

  xe = _pe_encode(x)
  hist = _make_hist_kernel()(ids)
  dpos, xs, bexp = _make_route_kernel()(ids, hist, xe)
  outs = _grouped_mlp(xs, bexp, w0p, b0r, w1c, b1r, w2c, b2r, w3c, b3r, w4p,
                      b4r)
  outw = _make_gather_kernel()(dpos, outs)
  return _narrow(outw) + 1e-3


# speedup vs baseline: 91.9429x; 1.3367x over previous
import functools

import jax
import jax.numpy as jnp
from jax import lax
from jax.experimental import pallas as pl
from jax.experimental.pallas import tpu as pltpu
from jax.experimental.pallas import tpu_sc as plsc

N_TOKENS = 131072
NUM_CLUSTERS = 16
HIDDEN = 256
OUT_DIM = 56
PE_LEVELS = 10
IN_DIM = 3 + 3 * 2 * PE_LEVELS
DW = 128

B_T = 512
N_PAD = N_TOKENS + NUM_CLUSTERS * B_T
NB = N_PAD // B_T
NB_PAD = ((NB + 15) // 16) * 16

NW = 32
CHUNK = N_TOKENS // NW
CROWS = CHUNK // DW

@functools.lru_cache(maxsize=None)
def _sc_mesh_opts():
  return dict(
      mesh=plsc.VectorSubcoreMesh(core_axis_name="c", subcore_axis_name="s"),
      compiler_params=pltpu.CompilerParams(needs_layout_passes=False),
  )


def _wid():
  return lax.axis_index("s") * 2 + lax.axis_index("c")


def _pe_body(x_ref, out_ref):
  xb = x_ref[...]
  rows = xb.shape[0]
  col = lax.broadcasted_iota(jnp.int32, (rows, DW), 1)
  col1 = lax.broadcasted_iota(jnp.int32, (3, DW), 1)
  drow = lax.broadcasted_iota(jnp.int32, (3, DW), 0)
  r = (col1 - 3) % 6
  dmap = jnp.where(col1 < 3, col1, r % 3)
  sel = jnp.where(dmap == drow, 1.0, 0.0).astype(jnp.float32)
  X = jnp.dot(xb, sel, preferred_element_type=jnp.float32)
  lvl = lax.broadcasted_iota(jnp.int32, (rows, DW), 1)
  lvl = jnp.maximum(lvl - 3, 0) // 6
  freqpow = jnp.exp2(lvl.astype(jnp.float32))
  is_cos = ((col - 3) % 6) >= 3
  phase = jnp.where(is_cos, 0.5, 0.0).astype(jnp.float32)
  u = X - 2.0 * jnp.round(X * 0.5)
  f = u * freqpow
  w = f + phase
  t = w - 2.0 * jnp.round(w * 0.5)
  a = jnp.abs(t)
  m = jnp.minimum(a, 1.0 - a)
  s = jnp.where(t < 0, -m, m)
  z = s * s
  p = jnp.float32(-7.3704309e-03)
  p = p * z + jnp.float32(8.2145887e-02)
  p = p * z - jnp.float32(5.9926453e-01)
  p = p * z + jnp.float32(2.5501640e+00)
  p = p * z - jnp.float32(5.1677128e+00)
  p = p * z + jnp.float32(3.1415927e+00)
  Z = s * p
  out_ref[...] = jnp.where(col < 3, X, jnp.where(col < IN_DIM, Z, 0.0))


def _pe_encode(x):
  rows = 1024
  return pl.pallas_call(
      _pe_body,
      grid=(N_TOKENS // rows,),
      in_specs=[pl.BlockSpec((rows, 3), lambda b: (b, 0))],
      out_specs=pl.BlockSpec((rows, DW), lambda b: (b, 0)),
      out_shape=jax.ShapeDtypeStruct((N_TOKENS, DW), jnp.float32),
  )(x)


@functools.lru_cache(maxsize=None)
def _make_hist_kernel():
  return pl.kernel(
      _hist_body,
      out_type=jax.ShapeDtypeStruct((NW, 16), jnp.int32),
      scratch_types=[
          pltpu.VMEM((CHUNK,), jnp.int32),
          pltpu.VMEM((16,), jnp.int32),
      ],
      **_sc_mesh_opts(),
  )


def _hist_body(ids_hbm, hist_o, ids_v, hist_v):
  wid = _wid()
  pltpu.sync_copy(ids_hbm.at[pl.ds(wid * CHUNK, CHUNK)], ids_v)
  hist_v[...] = jnp.zeros((16,), jnp.int32)
  ones = jnp.ones((16,), jnp.int32)

  def body(k, _):
    c = ids_v[pl.ds(k * 16, 16)]
    plsc.addupdate_scatter(hist_v, [c], ones)
    return 0

  lax.fori_loop(0, CHUNK // 16, body, 0)
  pltpu.sync_copy(hist_v, hist_o.at[wid])


@functools.lru_cache(maxsize=None)
def _make_pos_kernel():
  return pl.kernel(
      _pos_body,
      out_type=(
          jax.ShapeDtypeStruct((N_TOKENS // DW, DW), jnp.int32),
          jax.ShapeDtypeStruct((NB_PAD,), jnp.int32),
      ),
      scratch_types=[
          pltpu.VMEM((CHUNK,), jnp.int32),
          pltpu.VMEM((NW, 16), jnp.int32),
          pltpu.VMEM((16,), jnp.int32),
          pltpu.VMEM((16,), jnp.int32),
          pltpu.VMEM((16,), jnp.int32),
          pltpu.VMEM((16,), jnp.int32),
          pltpu.VMEM((CROWS, DW), jnp.int32),
          pltpu.VMEM((NB_PAD,), jnp.int32),
      ],
      **_sc_mesh_opts(),
  )


def _pos_body(ids_hbm, hist_hbm, dpos_o, bexp_o,
              ids_v, hist_v, base_v, seg_v, tmp_v, tmp2_v, pos_v, bexp_v):
  wid = _wid()
  i16 = lax.iota(jnp.int32, 16)
  ones = jnp.ones((16,), jnp.int32)
  zeros = jnp.zeros((16,), jnp.int32)

  pltpu.sync_copy(ids_hbm.at[pl.ds(wid * CHUNK, CHUNK)], ids_v)
  pltpu.sync_copy(hist_hbm, hist_v)

  totals = zeros
  pre = zeros
  for t in range(NW):
    row = hist_v[t]
    totals = totals + row
    pre = pre + jnp.where(t < wid, row, zeros)
  pt = (totals + (B_T - 1)) & ~(B_T - 1)
  seg = plsc.cumsum(pt) - pt
  base_v[...] = seg + pre
  seg_v[...] = seg

  def body(k, _):
    c = ids_v[pl.ds(k * 16, 16)]
    s, v = plsc.sort_key_val(c, i16)
    tmp_v[...] = s
    sh = plsc.load_gather(tmp_v, [jnp.maximum(i16 - 1, 0)])
    bnd = (i16 == 0) | (s != sh)
    st = plsc.cummax(jnp.where(bnd, i16, 0))
    plsc.store_scatter(tmp2_v, [v], i16 - st)
    rank = tmp2_v[...]
    g = plsc.load_gather(base_v, [c])
    pos = g + rank
    plsc.addupdate_scatter(base_v, [c], ones)
    pos_v[k // 8, pl.ds((k % 8) * 16, 16)] = pos
    return 0

  lax.fori_loop(0, CHUNK // 16, body, 0)

  pltpu.sync_copy(pos_v, dpos_o.at[pl.ds(wid * CROWS, CROWS)])

  @pl.when(wid == 0)
  def _():
    def bbody(i, _):
      p16 = i16 + i * 16
      acc = jnp.full((16,), -1, jnp.int32)
      for cl in range(NUM_CLUSTERS):
        sv = plsc.load_gather(seg_v, [jnp.full((16,), cl, jnp.int32)])
        acc = acc + jnp.where(p16 * B_T >= sv, 1, 0)
      bexp_v[pl.ds(i * 16, 16)] = acc
      return 0

    lax.fori_loop(0, NB_PAD // 16, bbody, 0)
    pltpu.sync_copy(bexp_v, bexp_o)


@functools.lru_cache(maxsize=None)
def _make_scatter_kernel():
  return pl.kernel(
      _scatter_body,
      out_type=jax.ShapeDtypeStruct((N_PAD, DW), jnp.float32),
      scratch_types=[
          pltpu.VMEM((CROWS, DW), jnp.int32),
          pltpu.VMEM((DW, DW), jnp.float32),
          pltpu.VMEM((DW, DW), jnp.float32),
          pltpu.SemaphoreType.DMA,
          pltpu.SemaphoreType.DMA,
      ],
      **_sc_mesh_opts(),
  )


def _scatter_body(dpos_hbm, xe_hbm, xs_o, pos_v, bufa, bufb, sema, semb):
  wid = _wid()
  pltpu.sync_copy(dpos_hbm.at[pl.ds(wid * CROWS, CROWS)], pos_v)

  row0 = wid * CHUNK
  pltpu.sync_copy(xe_hbm.at[pl.ds(row0, DW)], bufa)
  pltpu.async_copy(bufa, xs_o.at[pos_v.at[0]], sema)

  def sbody(j, _):
    @pl.when(j % 2 == 0)
    def _():
      pltpu.sync_copy(xe_hbm.at[pl.ds(row0 + (j + 1) * DW, DW)], bufb)
      pltpu.async_copy(bufb, xs_o.at[pos_v.at[j + 1]], semb)
      pltpu.make_async_copy(bufa, xs_o.at[pos_v.at[j]], sema).wait()

    @pl.when(j % 2 == 1)
    def _():
      pltpu.sync_copy(xe_hbm.at[pl.ds(row0 + (j + 1) * DW, DW)], bufa)
      pltpu.async_copy(bufa, xs_o.at[pos_v.at[j + 1]], sema)
      pltpu.make_async_copy(bufb, xs_o.at[pos_v.at[j]], semb).wait()

    return 0

  lax.fori_loop(0, CROWS - 1, sbody, 0)
  pltpu.make_async_copy(bufb, xs_o.at[pos_v.at[CROWS - 1]], semb).wait()


def _mlp_body(bexp_ref, xs_ref, w0, b0, w1, b1, w2, b2, w3, b3, w4, b4,
              out_ref):
  del bexp_ref
  f32 = jnp.float32
  bf = jnp.bfloat16
  h = xs_ref[...].astype(bf)
  h = jnp.tanh(jnp.dot(h, w0[0], preferred_element_type=f32) + b0[0])
  h = jnp.tanh(jnp.dot(h.astype(bf), w1[0], preferred_element_type=f32) + b1[0])
  h = jnp.tanh(jnp.dot(h.astype(bf), w2[0], preferred_element_type=f32) + b2[0])
  h = jnp.tanh(jnp.dot(h.astype(bf), w3[0], preferred_element_type=f32) + b3[0])
  out_ref[...] = jnp.dot(h.astype(bf), w4[0], preferred_element_type=f32) + b4[0]


def _grouped_mlp(xs, bexp, w0p, b0r, w1, b1r, w2, b2r, w3, b3r, w4p, b4r):
  def xmap(b, bexp_ref):
    del bexp_ref
    return (b, 0)

  def wmap(b, bexp_ref):
    return (bexp_ref[b], 0, 0)

  grid_spec = pltpu.PrefetchScalarGridSpec(
      num_scalar_prefetch=1,
      grid=(NB,),
      in_specs=[
          pl.BlockSpec((B_T, DW), xmap),
          pl.BlockSpec((1, DW, HIDDEN), wmap),
          pl.BlockSpec((1, 1, HIDDEN), wmap),
          pl.BlockSpec((1, HIDDEN, HIDDEN), wmap),
          pl.BlockSpec((1, 1, HIDDEN), wmap),
          pl.BlockSpec((1, HIDDEN, HIDDEN), wmap),
          pl.BlockSpec((1, 1, HIDDEN), wmap),
          pl.BlockSpec((1, HIDDEN, HIDDEN), wmap),
          pl.BlockSpec((1, 1, HIDDEN), wmap),
          pl.BlockSpec((1, HIDDEN, DW), wmap),
          pl.BlockSpec((1, 1, DW), wmap),
      ],
      out_specs=pl.BlockSpec((B_T, DW), xmap),
  )
  return pl.pallas_call(
      _mlp_body,
      grid_spec=grid_spec,
      out_shape=jax.ShapeDtypeStruct((N_PAD, DW), jnp.float32),
      compiler_params=pltpu.CompilerParams(
          dimension_semantics=("arbitrary",),
      ),
  )(bexp, xs, w0p, b0r, w1, b1r, w2, b2r, w3, b3r, w4p, b4r)


@functools.lru_cache(maxsize=None)
def _make_gather_kernel():
  return pl.kernel(
      _gather_body,
      out_type=jax.ShapeDtypeStruct((N_TOKENS, DW), jnp.float32),
      scratch_types=[
          pltpu.VMEM((CROWS, DW), jnp.int32),
          pltpu.VMEM((DW, DW), jnp.float32),
          pltpu.VMEM((DW, DW), jnp.float32),
          pltpu.SemaphoreType.DMA,
          pltpu.SemaphoreType.DMA,
      ],
      **_sc_mesh_opts(),
  )


def _gather_body(dpos_hbm, outs_hbm, outw_o, pos_v, bufa, bufb, sema, semb):
  wid = _wid()
  pltpu.sync_copy(dpos_hbm.at[pl.ds(wid * CROWS, CROWS)], pos_v)
  row0 = wid * CHUNK

  cpa = pltpu.async_copy(outs_hbm.at[pos_v.at[0]], bufa, sema)

  def gbody2(j, _):
    @pl.when(j % 2 == 0)
    def _():
      cpb = pltpu.async_copy(outs_hbm.at[pos_v.at[j + 1]], bufb, semb)
      pltpu.sync_copy(bufa, outw_o.at[pl.ds(row0 + j * DW, DW)])
      cpb.wait()

    @pl.when(j % 2 == 1)
    def _():
      cpa2 = pltpu.async_copy(outs_hbm.at[pos_v.at[j + 1]], bufa, sema)
      pltpu.sync_copy(bufb, outw_o.at[pl.ds(row0 + j * DW, DW)])
      cpa2.wait()

    return 0

  cpa.wait()
  lax.fori_loop(0, CROWS - 1, gbody2, 0)
  last = CROWS - 1
  buf_last = bufa if (last % 2 == 0) else bufb
  pltpu.sync_copy(buf_last, outw_o.at[pl.ds(row0 + last * DW, DW)])


def _narrow_body(in_ref, out_ref):
  out_ref[...] = in_ref[:, :OUT_DIM]


def _narrow(outw):
  rows = 1024
  return pl.pallas_call(
      _narrow_body,
      grid=(N_TOKENS // rows,),
      in_specs=[pl.BlockSpec((rows, DW), lambda b: (b, 0))],
      out_specs=pl.BlockSpec((rows, OUT_DIM), lambda b: (b, 0)),
      out_shape=jax.ShapeDtypeStruct((N_TOKENS, OUT_DIM), jnp.float32),
  )(outw)


def kernel(x, cluster_ids, W0, b0, W1, b1, W2, b2, W3, b3, W4, b4):
  ids = cluster_ids.astype(jnp.int32)

  w0p = jnp.pad(W0, ((0, 0), (0, DW - IN_DIM), (0, 0))).astype(jnp.bfloat16)
  w4p = jnp.pad(W4, ((0, 0), (0, 0), (0, DW - OUT_DIM))).astype(jnp.bfloat16)
  w1c = W1.astype(jnp.bfloat16)
  w2c = W2.astype(jnp.bfloat16)
  w3c = W3.astype(jnp.bfloat16)
  b4p = jnp.pad(b4, ((0, 0), (0, DW - OUT_DIM)))
  b0r = b0.reshape(NUM_CLUSTERS, 1, HIDDEN)
  b1r = b1.reshape(NUM_CLUSTERS, 1, HIDDEN)
  b2r = b2.reshape(NUM_CLUSTERS, 1, HIDDEN)
  b3r = b3.reshape(NUM_CLUSTERS, 1, HIDDEN)
  b4r = b4p.reshape(NUM_CLUSTERS, 1, DW)

  hist = _make_hist_kernel()(ids)
  dpos, bexp = _make_pos_kernel()(ids, hist)
  xe = _pe_encode(x)
  xs = _make_scatter_kernel()(dpos, xe)
  outs = _grouped_mlp(xs, bexp, w0p, b0r, w1c, b1r, w2c, b2r, w3c, b3r, w4p,
                      b4r)
  outw = _make_gather_kernel()(dpos, outs)
  return _narrow(outw)


# speedup vs baseline: 98.3259x; 1.0694x over previous
import functools

import jax
import jax.numpy as jnp
from jax import lax
from jax.experimental import pallas as pl
from jax.experimental.pallas import tpu as pltpu
from jax.experimental.pallas import tpu_sc as plsc

N_TOKENS = 131072
NUM_CLUSTERS = 16
HIDDEN = 256
OUT_DIM = 56
PE_LEVELS = 10
IN_DIM = 3 + 3 * 2 * PE_LEVELS
DW = 128

B_T = 512
N_PAD = N_TOKENS + NUM_CLUSTERS * B_T
NB = N_PAD // B_T
NB_PAD = ((NB + 15) // 16) * 16

NW = 32
CHUNK = N_TOKENS // NW
CROWS = CHUNK // DW

@functools.lru_cache(maxsize=None)
def _sc_mesh_opts():
  return dict(
      mesh=plsc.VectorSubcoreMesh(core_axis_name="c", subcore_axis_name="s"),
      compiler_params=pltpu.CompilerParams(needs_layout_passes=False),
  )


def _wid():
  return lax.axis_index("s") * 2 + lax.axis_index("c")


def _pe_body(x_ref, out_ref):
  xb = x_ref[...]
  rows = xb.shape[0]
  col = lax.broadcasted_iota(jnp.int32, (rows, DW), 1)
  col1 = lax.broadcasted_iota(jnp.int32, (3, DW), 1)
  drow = lax.broadcasted_iota(jnp.int32, (3, DW), 0)
  r = (col1 - 3) % 6
  dmap = jnp.where(col1 < 3, col1, r % 3)
  sel = jnp.where(dmap == drow, 1.0, 0.0).astype(jnp.float32)
  X = jnp.dot(xb, sel, preferred_element_type=jnp.float32)
  lvl = lax.broadcasted_iota(jnp.int32, (rows, DW), 1)
  lvl = jnp.maximum(lvl - 3, 0) // 6
  freqpow = jnp.exp2(lvl.astype(jnp.float32))
  is_cos = ((col - 3) % 6) >= 3
  phase = jnp.where(is_cos, 0.5, 0.0).astype(jnp.float32)
  u = X - 2.0 * jnp.round(X * 0.5)
  f = u * freqpow
  w = f + phase
  t = w - 2.0 * jnp.round(w * 0.5)
  a = jnp.abs(t)
  m = jnp.minimum(a, 1.0 - a)
  s = jnp.where(t < 0, -m, m)
  z = s * s
  p = jnp.float32(-7.3704309e-03)
  p = p * z + jnp.float32(8.2145887e-02)
  p = p * z - jnp.float32(5.9926453e-01)
  p = p * z + jnp.float32(2.5501640e+00)
  p = p * z - jnp.float32(5.1677128e+00)
  p = p * z + jnp.float32(3.1415927e+00)
  Z = s * p
  out_ref[...] = jnp.where(col < 3, X, jnp.where(col < IN_DIM, Z, 0.0))


def _pe_encode(x):
  rows = 2048
  return pl.pallas_call(
      _pe_body,
      grid=(N_TOKENS // rows,),
      in_specs=[pl.BlockSpec((rows, 3), lambda b: (b, 0))],
      out_specs=pl.BlockSpec((rows, DW), lambda b: (b, 0)),
      out_shape=jax.ShapeDtypeStruct((N_TOKENS, DW), jnp.float32),
  )(x)


@functools.lru_cache(maxsize=None)
def _make_hist_kernel():
  return pl.kernel(
      _hist_body,
      out_type=jax.ShapeDtypeStruct((NW, 16), jnp.int32),
      scratch_types=[
          pltpu.VMEM((CHUNK,), jnp.int32),
          pltpu.VMEM((16,), jnp.int32),
      ],
      **_sc_mesh_opts(),
  )


def _hist_body(ids_hbm, hist_o, ids_v, hist_v):
  wid = _wid()
  pltpu.sync_copy(ids_hbm.at[pl.ds(wid * CHUNK, CHUNK)], ids_v)
  hist_v[...] = jnp.zeros((16,), jnp.int32)
  ones = jnp.ones((16,), jnp.int32)

  def body(k, _):
    c = ids_v[pl.ds(k * 16, 16)]
    plsc.addupdate_scatter(hist_v, [c], ones)
    return 0

  lax.fori_loop(0, CHUNK // 16, body, 0)
  pltpu.sync_copy(hist_v, hist_o.at[wid])


@functools.lru_cache(maxsize=None)
def _make_pos_kernel():
  return pl.kernel(
      _pos_body,
      out_type=(
          jax.ShapeDtypeStruct((N_TOKENS // DW, DW), jnp.int32),
          jax.ShapeDtypeStruct((NB_PAD,), jnp.int32),
      ),
      scratch_types=[
          pltpu.VMEM((CHUNK,), jnp.int32),
          pltpu.VMEM((NW, 16), jnp.int32),
          pltpu.VMEM((16,), jnp.int32),
          pltpu.VMEM((16,), jnp.int32),
          pltpu.VMEM((16,), jnp.int32),
          pltpu.VMEM((16,), jnp.int32),
          pltpu.VMEM((CROWS, DW), jnp.int32),
          pltpu.VMEM((NB_PAD,), jnp.int32),
      ],
      **_sc_mesh_opts(),
  )


def _pos_body(ids_hbm, hist_hbm, dpos_o, bexp_o,
              ids_v, hist_v, base_v, seg_v, tmp_v, tmp2_v, pos_v, bexp_v):
  wid = _wid()
  i16 = lax.iota(jnp.int32, 16)
  ones = jnp.ones((16,), jnp.int32)
  zeros = jnp.zeros((16,), jnp.int32)

  pltpu.sync_copy(ids_hbm.at[pl.ds(wid * CHUNK, CHUNK)], ids_v)
  pltpu.sync_copy(hist_hbm, hist_v)

  totals = zeros
  pre = zeros
  for t in range(NW):
    row = hist_v[t]
    totals = totals + row
    pre = pre + jnp.where(t < wid, row, zeros)
  pt = (totals + (B_T - 1)) & ~(B_T - 1)
  seg = plsc.cumsum(pt) - pt
  base_v[...] = seg + pre
  seg_v[...] = seg

  def body(k, _):
    c = ids_v[pl.ds(k * 16, 16)]
    s, v = plsc.sort_key_val(c, i16)
    tmp_v[...] = s
    sh = plsc.load_gather(tmp_v, [jnp.maximum(i16 - 1, 0)])
    bnd = (i16 == 0) | (s != sh)
    st = plsc.cummax(jnp.where(bnd, i16, 0))
    plsc.store_scatter(tmp2_v, [v], i16 - st)
    rank = tmp2_v[...]
    g = plsc.load_gather(base_v, [c])
    pos = g + rank
    plsc.addupdate_scatter(base_v, [c], ones)
    pos_v[k // 8, pl.ds((k % 8) * 16, 16)] = pos
    return 0

  lax.fori_loop(0, CHUNK // 16, body, 0)

  pltpu.sync_copy(pos_v, dpos_o.at[pl.ds(wid * CROWS, CROWS)])

  @pl.when(wid == 0)
  def _():
    def bbody(i, _):
      p16 = i16 + i * 16
      acc = jnp.full((16,), -1, jnp.int32)
      for cl in range(NUM_CLUSTERS):
        sv = plsc.load_gather(seg_v, [jnp.full((16,), cl, jnp.int32)])
        acc = acc + jnp.where(p16 * B_T >= sv, 1, 0)
      bexp_v[pl.ds(i * 16, 16)] = acc
      return 0

    lax.fori_loop(0, NB_PAD // 16, bbody, 0)
    pltpu.sync_copy(bexp_v, bexp_o)


@functools.lru_cache(maxsize=None)
def _make_scatter_kernel():
  return pl.kernel(
      _scatter_body,
      out_type=jax.ShapeDtypeStruct((N_PAD, DW), jnp.float32),
      scratch_types=[
          pltpu.VMEM((CROWS, DW), jnp.int32),
          pltpu.VMEM((DW, DW), jnp.float32),
          pltpu.VMEM((DW, DW), jnp.float32),
          pltpu.SemaphoreType.DMA,
          pltpu.SemaphoreType.DMA,
      ],
      **_sc_mesh_opts(),
  )


def _scatter_body(dpos_hbm, xe_hbm, xs_o, pos_v, bufa, bufb, sema, semb):
  wid = _wid()
  pltpu.sync_copy(dpos_hbm.at[pl.ds(wid * CROWS, CROWS)], pos_v)

  row0 = wid * CHUNK
  pltpu.sync_copy(xe_hbm.at[pl.ds(row0, DW)], bufa)
  pltpu.async_copy(bufa, xs_o.at[pos_v.at[0]], sema)

  def sbody(j, _):
    @pl.when(j % 2 == 0)
    def _():
      pltpu.sync_copy(xe_hbm.at[pl.ds(row0 + (j + 1) * DW, DW)], bufb)
      pltpu.async_copy(bufb, xs_o.at[pos_v.at[j + 1]], semb)
      pltpu.make_async_copy(bufa, xs_o.at[pos_v.at[j]], sema).wait()

    @pl.when(j % 2 == 1)
    def _():
      pltpu.sync_copy(xe_hbm.at[pl.ds(row0 + (j + 1) * DW, DW)], bufa)
      pltpu.async_copy(bufa, xs_o.at[pos_v.at[j + 1]], sema)
      pltpu.make_async_copy(bufb, xs_o.at[pos_v.at[j]], semb).wait()

    return 0

  lax.fori_loop(0, CROWS - 1, sbody, 0)
  pltpu.make_async_copy(bufb, xs_o.at[pos_v.at[CROWS - 1]], semb).wait()


def _mlp_body(bexp_ref, xs_ref, w_ref, b_ref, out_ref):
  del bexp_ref
  f32 = jnp.float32
  bf = jnp.bfloat16
  W = w_ref[0]
  B = b_ref[0]
  h = xs_ref[...].astype(bf)
  h = jnp.tanh(jnp.dot(h, W[0, :DW, :], preferred_element_type=f32) + B[0:1])
  h = jnp.tanh(jnp.dot(h.astype(bf), W[1], preferred_element_type=f32) + B[1:2])
  h = jnp.tanh(jnp.dot(h.astype(bf), W[2], preferred_element_type=f32) + B[2:3])
  h = jnp.tanh(jnp.dot(h.astype(bf), W[3], preferred_element_type=f32) + B[3:4])
  out_ref[...] = (jnp.dot(h.astype(bf), W[4, :, :DW],
                          preferred_element_type=f32) + B[4:5, :DW])


def _grouped_mlp(xs, bexp, wstk, bstk):
  def xmap(b, bexp_ref):
    del bexp_ref
    return (b, 0)

  def wmap(b, bexp_ref):
    return (bexp_ref[b], 0, 0, 0)

  def bmap(b, bexp_ref):
    return (bexp_ref[b], 0, 0)

  grid_spec = pltpu.PrefetchScalarGridSpec(
      num_scalar_prefetch=1,
      grid=(NB,),
      in_specs=[
          pl.BlockSpec((B_T, DW), xmap),
          pl.BlockSpec((1, 5, HIDDEN, HIDDEN), wmap),
          pl.BlockSpec((1, 5, HIDDEN), bmap),
      ],
      out_specs=pl.BlockSpec((B_T, DW), xmap),
  )
  return pl.pallas_call(
      _mlp_body,
      grid_spec=grid_spec,
      out_shape=jax.ShapeDtypeStruct((N_PAD, DW), jnp.float32),
      compiler_params=pltpu.CompilerParams(
          dimension_semantics=("arbitrary",),
      ),
  )(bexp, xs, wstk, bstk)


@functools.lru_cache(maxsize=None)
def _make_gather_kernel():
  return pl.kernel(
      _gather_body,
      out_type=jax.ShapeDtypeStruct((N_TOKENS, DW), jnp.float32),
      scratch_types=[
          pltpu.VMEM((CROWS, DW), jnp.int32),
          pltpu.VMEM((DW, DW), jnp.float32),
          pltpu.VMEM((DW, DW), jnp.float32),
          pltpu.SemaphoreType.DMA,
          pltpu.SemaphoreType.DMA,
      ],
      **_sc_mesh_opts(),
  )


def _gather_body(dpos_hbm, outs_hbm, outw_o, pos_v, bufa, bufb, sema, semb):
  wid = _wid()
  pltpu.sync_copy(dpos_hbm.at[pl.ds(wid * CROWS, CROWS)], pos_v)
  row0 = wid * CHUNK

  cpa = pltpu.async_copy(outs_hbm.at[pos_v.at[0]], bufa, sema)

  def gbody2(j, _):
    @pl.when(j % 2 == 0)
    def _():
      cpb = pltpu.async_copy(outs_hbm.at[pos_v.at[j + 1]], bufb, semb)
      pltpu.sync_copy(bufa, outw_o.at[pl.ds(row0 + j * DW, DW)])
      cpb.wait()

    @pl.when(j % 2 == 1)
    def _():
      cpa2 = pltpu.async_copy(outs_hbm.at[pos_v.at[j + 1]], bufa, sema)
      pltpu.sync_copy(bufb, outw_o.at[pl.ds(row0 + j * DW, DW)])
      cpa2.wait()

    return 0

  cpa.wait()
  lax.fori_loop(0, CROWS - 1, gbody2, 0)
  last = CROWS - 1
  buf_last = bufa if (last % 2 == 0) else bufb
  pltpu.sync_copy(buf_last, outw_o.at[pl.ds(row0 + last * DW, DW)])


def _narrow_body(in_ref, out_ref):
  out_ref[...] = in_ref[:, :OUT_DIM]


def _narrow(outw):
  rows = 1024
  return pl.pallas_call(
      _narrow_body,
      grid=(N_TOKENS // rows,),
      in_specs=[pl.BlockSpec((rows, DW), lambda b: (b, 0))],
      out_specs=pl.BlockSpec((rows, OUT_DIM), lambda b: (b, 0)),
      out_shape=jax.ShapeDtypeStruct((N_TOKENS, OUT_DIM), jnp.float32),
  )(outw)


def kernel(x, cluster_ids, W0, b0, W1, b1, W2, b2, W3, b3, W4, b4):
  ids = cluster_ids.astype(jnp.int32)

  w0p = jnp.pad(W0, ((0, 0), (0, HIDDEN - IN_DIM), (0, 0)))
  w4p = jnp.pad(W4, ((0, 0), (0, 0), (0, HIDDEN - OUT_DIM)))
  wstk = jnp.stack([w0p, W1, W2, W3, w4p], axis=1).astype(jnp.bfloat16)
  b4p = jnp.pad(b4, ((0, 0), (0, HIDDEN - OUT_DIM)))
  bstk = jnp.stack([b0, b1, b2, b3, b4p], axis=1)

  hist = _make_hist_kernel()(ids)
  dpos, bexp = _make_pos_kernel()(ids, hist)
  xe = _pe_encode(x)
  xs = _make_scatter_kernel()(dpos, xe)
  outs = _grouped_mlp(xs, bexp, wstk, bstk)
  outw = _make_gather_kernel()(dpos, outs)
  return _narrow(outw)


# speedup vs baseline: 101.3019x; 1.0303x over previous
import functools

import jax
import jax.numpy as jnp
from jax import lax
from jax.experimental import pallas as pl
from jax.experimental.pallas import tpu as pltpu
from jax.experimental.pallas import tpu_sc as plsc

N_TOKENS = 131072
NUM_CLUSTERS = 16
HIDDEN = 256
OUT_DIM = 56
PE_LEVELS = 10
IN_DIM = 3 + 3 * 2 * PE_LEVELS
DW = 128

B_T = 512
N_PAD = N_TOKENS + NUM_CLUSTERS * B_T
NB = N_PAD // B_T
NB_PAD = ((NB + 15) // 16) * 16

NW = 32
CHUNK = N_TOKENS // NW
RB = 256
CROWS = CHUNK // RB

@functools.lru_cache(maxsize=None)
def _sc_mesh_opts():
  return dict(
      mesh=plsc.VectorSubcoreMesh(core_axis_name="c", subcore_axis_name="s"),
      compiler_params=pltpu.CompilerParams(needs_layout_passes=False),
  )


def _wid():
  return lax.axis_index("s") * 2 + lax.axis_index("c")


def _pe_body(x_ref, out_ref):
  xb = x_ref[...]
  rows = xb.shape[0]
  col = lax.broadcasted_iota(jnp.int32, (rows, DW), 1)
  col1 = lax.broadcasted_iota(jnp.int32, (3, DW), 1)
  drow = lax.broadcasted_iota(jnp.int32, (3, DW), 0)
  r = (col1 - 3) % 6
  dmap = jnp.where(col1 < 3, col1, r % 3)
  sel = jnp.where(dmap == drow, 1.0, 0.0).astype(jnp.float32)
  X = jnp.dot(xb, sel, preferred_element_type=jnp.float32)
  lvl = lax.broadcasted_iota(jnp.int32, (rows, DW), 1)
  lvl = jnp.maximum(lvl - 3, 0) // 6
  freqpow = jnp.exp2(lvl.astype(jnp.float32))
  is_cos = ((col - 3) % 6) >= 3
  phase = jnp.where(is_cos, 0.5, 0.0).astype(jnp.float32)
  u = X - 2.0 * jnp.round(X * 0.5)
  f = u * freqpow
  w = f + phase
  t = w - 2.0 * jnp.round(w * 0.5)
  a = jnp.abs(t)
  m = jnp.minimum(a, 1.0 - a)
  s = jnp.where(t < 0, -m, m)
  z = s * s
  p = jnp.float32(-7.3704309e-03)
  p = p * z + jnp.float32(8.2145887e-02)
  p = p * z - jnp.float32(5.9926453e-01)
  p = p * z + jnp.float32(2.5501640e+00)
  p = p * z - jnp.float32(5.1677128e+00)
  p = p * z + jnp.float32(3.1415927e+00)
  Z = s * p
  out_ref[...] = jnp.where(col < 3, X, jnp.where(col < IN_DIM, Z, 0.0))


def _pe_encode(x):
  rows = 2048
  return pl.pallas_call(
      _pe_body,
      grid=(N_TOKENS // rows,),
      in_specs=[pl.BlockSpec((rows, 3), lambda b: (b, 0))],
      out_specs=pl.BlockSpec((rows, DW), lambda b: (b, 0)),
      out_shape=jax.ShapeDtypeStruct((N_TOKENS, DW), jnp.float32),
  )(x)


@functools.lru_cache(maxsize=None)
def _make_hist_kernel():
  return pl.kernel(
      _hist_body,
      out_type=jax.ShapeDtypeStruct((NW, 16), jnp.int32),
      scratch_types=[
          pltpu.VMEM((CHUNK,), jnp.int32),
          pltpu.VMEM((16,), jnp.int32),
      ],
      **_sc_mesh_opts(),
  )


def _hist_body(ids_hbm, hist_o, ids_v, hist_v):
  wid = _wid()
  pltpu.sync_copy(ids_hbm.at[pl.ds(wid * CHUNK, CHUNK)], ids_v)
  hist_v[...] = jnp.zeros((16,), jnp.int32)
  ones = jnp.ones((16,), jnp.int32)

  def body(k, _):
    c = ids_v[pl.ds(k * 16, 16)]
    plsc.addupdate_scatter(hist_v, [c], ones)
    return 0

  lax.fori_loop(0, CHUNK // 16, body, 0)
  pltpu.sync_copy(hist_v, hist_o.at[wid])


@functools.lru_cache(maxsize=None)
def _make_pos_kernel():
  return pl.kernel(
      _pos_body,
      out_type=(
          jax.ShapeDtypeStruct((N_TOKENS,), jnp.int32),
          jax.ShapeDtypeStruct((NB_PAD,), jnp.int32),
      ),
      scratch_types=[
          pltpu.VMEM((CHUNK,), jnp.int32),
          pltpu.VMEM((NW, 16), jnp.int32),
          pltpu.VMEM((16,), jnp.int32),
          pltpu.VMEM((16,), jnp.int32),
          pltpu.VMEM((16,), jnp.int32),
          pltpu.VMEM((16,), jnp.int32),
          pltpu.VMEM((CHUNK,), jnp.int32),
          pltpu.VMEM((NB_PAD,), jnp.int32),
      ],
      **_sc_mesh_opts(),
  )


def _pos_body(ids_hbm, hist_hbm, dpos_o, bexp_o,
              ids_v, hist_v, base_v, seg_v, tmp_v, tmp2_v, pos_v, bexp_v):
  wid = _wid()
  i16 = lax.iota(jnp.int32, 16)
  ones = jnp.ones((16,), jnp.int32)
  zeros = jnp.zeros((16,), jnp.int32)

  pltpu.sync_copy(ids_hbm.at[pl.ds(wid * CHUNK, CHUNK)], ids_v)
  pltpu.sync_copy(hist_hbm, hist_v)

  totals = zeros
  pre = zeros
  for t in range(NW):
    row = hist_v[t]
    totals = totals + row
    pre = pre + jnp.where(t < wid, row, zeros)
  pt = (totals + (B_T - 1)) & ~(B_T - 1)
  seg = plsc.cumsum(pt) - pt
  base_v[...] = seg + pre
  seg_v[...] = seg

  def body(k, _):
    c = ids_v[pl.ds(k * 16, 16)]
    s, v = plsc.sort_key_val(c, i16)
    tmp_v[...] = s
    sh = plsc.load_gather(tmp_v, [jnp.maximum(i16 - 1, 0)])
    bnd = (i16 == 0) | (s != sh)
    st = plsc.cummax(jnp.where(bnd, i16, 0))
    plsc.store_scatter(tmp2_v, [v], i16 - st)
    rank = tmp2_v[...]
    g = plsc.load_gather(base_v, [c])
    pos = g + rank
    plsc.addupdate_scatter(base_v, [c], ones)
    pos_v[pl.ds(k * 16, 16)] = pos
    return 0

  lax.fori_loop(0, CHUNK // 16, body, 0)

  pltpu.sync_copy(pos_v, dpos_o.at[pl.ds(wid * CHUNK, CHUNK)])

  @pl.when(wid == 0)
  def _():
    def bbody(i, _):
      p16 = i16 + i * 16
      acc = jnp.full((16,), -1, jnp.int32)
      for cl in range(NUM_CLUSTERS):
        sv = plsc.load_gather(seg_v, [jnp.full((16,), cl, jnp.int32)])
        acc = acc + jnp.where(p16 * B_T >= sv, 1, 0)
      bexp_v[pl.ds(i * 16, 16)] = acc
      return 0

    lax.fori_loop(0, NB_PAD // 16, bbody, 0)
    pltpu.sync_copy(bexp_v, bexp_o)


@functools.lru_cache(maxsize=None)
def _make_scatter_kernel():
  return pl.kernel(
      _scatter_body,
      out_type=jax.ShapeDtypeStruct((N_PAD, DW), jnp.float32),
      scratch_types=[
          pltpu.VMEM((CHUNK,), jnp.int32),
          pltpu.VMEM((RB, DW), jnp.float32),
          pltpu.VMEM((RB, DW), jnp.float32),
          pltpu.SemaphoreType.DMA,
          pltpu.SemaphoreType.DMA,
      ],
      **_sc_mesh_opts(),
  )


def _scatter_body(dpos_hbm, xe_hbm, xs_o, pos_v, bufa, bufb, sema, semb):
  wid = _wid()
  pltpu.sync_copy(dpos_hbm.at[pl.ds(wid * CHUNK, CHUNK)], pos_v)

  row0 = wid * CHUNK
  pltpu.sync_copy(xe_hbm.at[pl.ds(row0, RB)], bufa)
  pltpu.async_copy(bufa, xs_o.at[pos_v.at[pl.ds(0, RB)]], sema)

  def sbody(j, _):
    @pl.when(j % 2 == 0)
    def _():
      pltpu.sync_copy(xe_hbm.at[pl.ds(row0 + (j + 1) * RB, RB)], bufb)
      pltpu.async_copy(bufb, xs_o.at[pos_v.at[pl.ds((j + 1) * RB, RB)]], semb)
      pltpu.make_async_copy(bufa, xs_o.at[pos_v.at[pl.ds(j * RB, RB)]], sema).wait()

    @pl.when(j % 2 == 1)
    def _():
      pltpu.sync_copy(xe_hbm.at[pl.ds(row0 + (j + 1) * RB, RB)], bufa)
      pltpu.async_copy(bufa, xs_o.at[pos_v.at[pl.ds((j + 1) * RB, RB)]], sema)
      pltpu.make_async_copy(bufb, xs_o.at[pos_v.at[pl.ds(j * RB, RB)]], semb).wait()

    return 0

  lax.fori_loop(0, CROWS - 1, sbody, 0)
  pltpu.make_async_copy(bufb, xs_o.at[pos_v.at[pl.ds((CROWS - 1) * RB, RB)]], semb).wait()


def _mlp_body(bexp_ref, xs_ref, w_ref, b_ref, out_ref):
  del bexp_ref
  f32 = jnp.float32
  bf = jnp.bfloat16
  W = w_ref[0]
  B = b_ref[0]
  h = xs_ref[...].astype(bf)
  h = jnp.tanh(jnp.dot(h, W[0, :DW, :], preferred_element_type=f32) + B[0:1])
  h = jnp.tanh(jnp.dot(h.astype(bf), W[1], preferred_element_type=f32) + B[1:2])
  h = jnp.tanh(jnp.dot(h.astype(bf), W[2], preferred_element_type=f32) + B[2:3])
  h = jnp.tanh(jnp.dot(h.astype(bf), W[3], preferred_element_type=f32) + B[3:4])
  out_ref[...] = (jnp.dot(h.astype(bf), W[4, :, :DW],
                          preferred_element_type=f32) + B[4:5, :DW])


def _grouped_mlp(xs, bexp, wstk, bstk):
  def xmap(b, bexp_ref):
    del bexp_ref
    return (b, 0)

  def wmap(b, bexp_ref):
    return (bexp_ref[b], 0, 0, 0)

  def bmap(b, bexp_ref):
    return (bexp_ref[b], 0, 0)

  grid_spec = pltpu.PrefetchScalarGridSpec(
      num_scalar_prefetch=1,
      grid=(NB,),
      in_specs=[
          pl.BlockSpec((B_T, DW), xmap),
          pl.BlockSpec((1, 5, HIDDEN, HIDDEN), wmap),
          pl.BlockSpec((1, 5, HIDDEN), bmap),
      ],
      out_specs=pl.BlockSpec((B_T, DW), xmap),
  )
  return pl.pallas_call(
      _mlp_body,
      grid_spec=grid_spec,
      out_shape=jax.ShapeDtypeStruct((N_PAD, DW), jnp.float32),
      compiler_params=pltpu.CompilerParams(
          dimension_semantics=("arbitrary",),
      ),
  )(bexp, xs, wstk, bstk)


@functools.lru_cache(maxsize=None)
def _make_gather_kernel():
  return pl.kernel(
      _gather_body,
      out_type=jax.ShapeDtypeStruct((N_TOKENS, DW), jnp.float32),
      scratch_types=[
          pltpu.VMEM((CHUNK,), jnp.int32),
          pltpu.VMEM((RB, DW), jnp.float32),
          pltpu.VMEM((RB, DW), jnp.float32),
          pltpu.SemaphoreType.DMA,
          pltpu.SemaphoreType.DMA,
      ],
      **_sc_mesh_opts(),
  )


def _gather_body(dpos_hbm, outs_hbm, outw_o, pos_v, bufa, bufb, sema, semb):
  wid = _wid()
  pltpu.sync_copy(dpos_hbm.at[pl.ds(wid * CHUNK, CHUNK)], pos_v)
  row0 = wid * CHUNK

  cpa = pltpu.async_copy(outs_hbm.at[pos_v.at[pl.ds(0, RB)]], bufa, sema)

  def gbody2(j, _):
    @pl.when(j % 2 == 0)
    def _():
      cpb = pltpu.async_copy(outs_hbm.at[pos_v.at[pl.ds((j + 1) * RB, RB)]], bufb, semb)
      pltpu.sync_copy(bufa, outw_o.at[pl.ds(row0 + j * RB, RB)])
      cpb.wait()

    @pl.when(j % 2 == 1)
    def _():
      cpa2 = pltpu.async_copy(outs_hbm.at[pos_v.at[pl.ds((j + 1) * RB, RB)]], bufa, sema)
      pltpu.sync_copy(bufb, outw_o.at[pl.ds(row0 + j * RB, RB)])
      cpa2.wait()

    return 0

  cpa.wait()
  lax.fori_loop(0, CROWS - 1, gbody2, 0)
  last = CROWS - 1
  buf_last = bufa if (last % 2 == 0) else bufb
  pltpu.sync_copy(buf_last, outw_o.at[pl.ds(row0 + last * RB, RB)])


def _narrow_body(in_ref, out_ref):
  out_ref[...] = in_ref[:, :OUT_DIM]


def _narrow(outw):
  rows = 1024
  return pl.pallas_call(
      _narrow_body,
      grid=(N_TOKENS // rows,),
      in_specs=[pl.BlockSpec((rows, DW), lambda b: (b, 0))],
      out_specs=pl.BlockSpec((rows, OUT_DIM), lambda b: (b, 0)),
      out_shape=jax.ShapeDtypeStruct((N_TOKENS, OUT_DIM), jnp.float32),
  )(outw)


def kernel(x, cluster_ids, W0, b0, W1, b1, W2, b2, W3, b3, W4, b4):
  ids = cluster_ids.astype(jnp.int32)

  w0p = jnp.pad(W0, ((0, 0), (0, HIDDEN - IN_DIM), (0, 0)))
  w4p = jnp.pad(W4, ((0, 0), (0, 0), (0, HIDDEN - OUT_DIM)))
  wstk = jnp.stack([w0p, W1, W2, W3, w4p], axis=1).astype(jnp.bfloat16)
  b4p = jnp.pad(b4, ((0, 0), (0, HIDDEN - OUT_DIM)))
  bstk = jnp.stack([b0, b1, b2, b3, b4p], axis=1)

  hist = _make_hist_kernel()(ids)
  dpos, bexp = _make_pos_kernel()(ids, hist)
  xe = _pe_encode(x)
  xs = _make_scatter_kernel()(dpos, xe)
  outs = _grouped_mlp(xs, bexp, wstk, bstk)
  outw = _make_gather_kernel()(dpos, outs)
  return _narrow(outw)
